# Initial kernel scaffold; baseline (speedup 1.0000x reference)
import jax
import jax.numpy as jnp
from jax.experimental import pallas as pl


def kernel(h, sbf_rbf_w1, sbf_sph, W_down, W_bilinear, W_m_st, W_st, W_ts, scale_sbf, idx_s, idx_t, idx_swap, edge_nb_idx, edge_nb_ragged_idx):
    raise NotImplementedError("write your pallas kernel here")


# speedup vs baseline: 9.0644x; 9.0644x over previous
import functools

import numpy as np
import jax
import jax.numpy as jnp
from jax import lax
from jax.experimental import pallas as pl
from jax.experimental.pallas import tpu as pltpu

INV_SQRT_2 = 1.0 / 2.0 ** 0.5
Q = 16
SB = 16
NS = 7
K = 4


def _silu(x):
    return x * jax.nn.sigmoid(x)


def _hdown_body(h_ref, w_ref, o_ref):
    o_ref[...] = _silu(jnp.dot(h_ref[...], w_ref[...],
                               preferred_element_type=jnp.float32))


def _tc_hdown(h, W_down, bn=1000):
    n, d = h.shape
    q = W_down.shape[1]
    return pl.pallas_call(
        _hdown_body,
        grid=(n // bn,),
        in_specs=[pl.BlockSpec((bn, d), lambda i: (i, 0)),
                  pl.BlockSpec((d, q), lambda i: (0, 0))],
        out_specs=pl.BlockSpec((bn, q), lambda i: (i, 0)),
        out_shape=jax.ShapeDtypeStruct((n, q), jnp.float32),
    )(h, W_down)


def _edge_body(sph_ref, rbf_ref, ht_ref, rt_ref, s_ref, a_ref, b_ref, wb_ref,
               o_ref):
    tmp = jnp.dot(sph_ref[...], rt_ref[...], preferred_element_type=jnp.float32)
    c = jnp.dot(rbf_ref[...] * tmp, s_ref[...],
                preferred_element_type=jnp.float32)
    u = jnp.dot(ht_ref[...], a_ref[...], preferred_element_type=jnp.float32) \
        * jnp.dot(c, b_ref[...], preferred_element_type=jnp.float32)
    o_ref[...] = jnp.dot(u, wb_ref[...], preferred_element_type=jnp.float32)


def _tc_edge_bilinear(sph28, rbf112, ht, RT, S, A, B, Wb2, be=2000):
    e = sph28.shape[0]
    return pl.pallas_call(
        _edge_body,
        grid=(e // be,),
        in_specs=[pl.BlockSpec((be, K * NS), lambda i: (i, 0)),
                  pl.BlockSpec((be, SB * NS), lambda i: (i, 0)),
                  pl.BlockSpec((be, Q), lambda i: (i, 0)),
                  pl.BlockSpec(RT.shape, lambda i: (0, 0)),
                  pl.BlockSpec(S.shape, lambda i: (0, 0)),
                  pl.BlockSpec(A.shape, lambda i: (0, 0)),
                  pl.BlockSpec(B.shape, lambda i: (0, 0)),
                  pl.BlockSpec(Wb2.shape, lambda i: (0, 0))],
        out_specs=pl.BlockSpec((be, Q), lambda i: (i, 0)),
        out_shape=jax.ShapeDtypeStruct((e, Q), jnp.float32),
    )(sph28, rbf112, ht, RT, S, A, B, Wb2)


def _combine_body(a_ref, b_ref, o_ref):
    o_ref[...] = a_ref[...] + b_ref[...]


def _tc_combine(p0, p1, bn=1000):
    n, q = p0.shape
    return pl.pallas_call(
        _combine_body,
        grid=(n // bn,),
        in_specs=[pl.BlockSpec((bn, q), lambda i: (i, 0)),
                  pl.BlockSpec((bn, q), lambda i: (i, 0))],
        out_specs=pl.BlockSpec((bn, q), lambda i: (i, 0)),
        out_shape=jax.ShapeDtypeStruct((n, q), jnp.float32),
    )(p0, p1)


def _mst_body(gs_ref, gt_ref, w1_ref, w2_ref, o_ref):
    o_ref[...] = _silu(
        jnp.dot(gs_ref[...], w1_ref[...], preferred_element_type=jnp.float32)
        + jnp.dot(gt_ref[...], w2_ref[...], preferred_element_type=jnp.float32))


def _tc_mst(gs, gt, W1, W2, be=2000):
    e, q = gs.shape
    return pl.pallas_call(
        _mst_body,
        grid=(e // be,),
        in_specs=[pl.BlockSpec((be, q), lambda i: (i, 0)),
                  pl.BlockSpec((be, q), lambda i: (i, 0)),
                  pl.BlockSpec(W1.shape, lambda i: (0, 0)),
                  pl.BlockSpec(W2.shape, lambda i: (0, 0))],
        out_specs=pl.BlockSpec((be, q), lambda i: (i, 0)),
        out_shape=jax.ShapeDtypeStruct((e, q), jnp.float32),
    )(gs, gt, W1, W2)


def _final_body(x_ref, xs_ref, wst_ref, wts_ref, o_ref):
    st = _silu(jnp.dot(x_ref[...], wst_ref[...],
                       preferred_element_type=jnp.float32))
    ts = _silu(jnp.dot(xs_ref[...], wts_ref[...],
                       preferred_element_type=jnp.float32))
    o_ref[...] = (st + ts) * INV_SQRT_2


def _tc_final(x, xs, W_st, W_ts, be=2000):
    e, q = x.shape
    d = W_st.shape[1]
    return pl.pallas_call(
        _final_body,
        grid=(e // be,),
        in_specs=[pl.BlockSpec((be, q), lambda i: (i, 0)),
                  pl.BlockSpec((be, q), lambda i: (i, 0)),
                  pl.BlockSpec(W_st.shape, lambda i: (0, 0)),
                  pl.BlockSpec(W_ts.shape, lambda i: (0, 0))],
        out_specs=pl.BlockSpec((be, d), lambda i: (i, 0)),
        out_shape=jax.ShapeDtypeStruct((e, d), jnp.float32),
    )(x, xs, W_st, W_ts)


def _const_mats():
    RT = np.zeros((K * NS, SB * NS), np.float32)
    for k in range(K):
        for s in range(NS):
            for i in range(SB):
                RT[k * NS + s, i * NS + s] = 1.0
    S = np.zeros((SB * NS, SB), np.float32)
    for i in range(SB):
        for s in range(NS):
            S[i * NS + s, i] = 1.0
    A = np.zeros((Q, Q * SB), np.float32)
    B = np.zeros((SB, Q * SB), np.float32)
    for q in range(Q):
        for i in range(SB):
            A[q, q * SB + i] = 1.0
            B[i, q * SB + i] = 1.0
    return jnp.asarray(RT), jnp.asarray(S), jnp.asarray(A), jnp.asarray(B)


def kernel(h, sbf_rbf_w1, sbf_sph, W_down, W_bilinear, W_m_st, W_st, W_ts,
           scale_sbf, idx_s, idx_t, idx_swap, edge_nb_idx, edge_nb_ragged_idx):
    n = h.shape[0]
    e = sbf_rbf_w1.shape[0]

    RT, S, A, B = _const_mats()
    Wb2 = W_bilinear.reshape(Q * SB, Q)
    sph28 = sbf_sph.reshape(e, K * NS)
    rbf112 = sbf_rbf_w1.reshape(e, SB * NS)
    Wms = W_m_st[:Q] * scale_sbf
    Wmt = W_m_st[Q:] * scale_sbf

    h_down = _tc_hdown(h, W_down)
    ht = h_down[idx_t]
    hsbf = _tc_edge_bilinear(sph28, rbf112, ht, RT, S, A, B, Wb2)
    h_mp = jax.ops.segment_sum(hsbf, idx_s, num_segments=n)
    gs = h_mp[idx_s]
    gt = h_mp[idx_t]
    x = _tc_mst(gs, gt, Wms, Wmt)
    xs = x[idx_swap]
    return _tc_final(x, xs, W_st, W_ts)
